# baseline (speedup 1.0000x reference)
import functools

import jax
import jax.numpy as jnp
from jax import lax
from jax.experimental import pallas as pl
from jax.experimental.pallas import tpu as pltpu
from jax.experimental.pallas import tpu_sc as plsc

NC = 2
NS = 16
NW = NC * NS
L = 16


def _make_sc_kernel(B, S, D):
    rows_w = S // NW
    T = 8
    n_chunks = rows_w // T
    n_pairs = n_chunks // 2

    mesh = plsc.VectorSubcoreMesh(core_axis_name="c", subcore_axis_name="s")

    @functools.partial(
        pl.kernel,
        out_type=jax.ShapeDtypeStruct((B, S, D), jnp.float32),
        mesh=mesh,
        scratch_types=[
            [pltpu.VMEM((T, D), jnp.float32)] * 4,
            [pltpu.VMEM((T, D), jnp.float32)] * 2,
            [pltpu.SemaphoreType.DMA] * 4,
            [pltpu.SemaphoreType.DMA] * 4,
            [pltpu.SemaphoreType.DMA] * 2,
        ],
    )
    def sc_kernel(tok_hbm, pos_hbm, out_hbm, tv, pv, sin, sout, spos):
        wid = lax.axis_index("s") * NC + lax.axis_index("c")
        s0 = wid * rows_w

        def start_in(b, row, slot):
            pltpu.async_copy(tok_hbm.at[b, pl.ds(row, T), :], tv[slot],
                             sin[slot])

        def drain_in(slot):
            pltpu.make_async_copy(tok_hbm.at[0, pl.ds(0, T), :], tv[slot],
                                  sin[slot]).wait()

        def start_out(b, row, slot):
            pltpu.async_copy(tv[slot], out_hbm.at[b, pl.ds(row, T), :],
                             sout[slot])

        def drain_out(slot):
            pltpu.make_async_copy(tv[slot], out_hbm.at[0, pl.ds(0, T), :],
                                  sout[slot]).wait()

        def start_pos(row, pslot):
            pltpu.async_copy(pos_hbm.at[pl.ds(row, T), :], pv[pslot],
                             spos[pslot])

        def drain_pos(pslot):
            pltpu.make_async_copy(pos_hbm.at[pl.ds(0, T), :], pv[pslot],
                                  spos[pslot]).wait()

        start_in(0, s0, 0)
        start_in(1, s0, 1)
        start_pos(s0, 0)
        start_pos(s0 + T, 1)

        def pair_body(h, first, last):
            row0 = s0 + h * (2 * T)
            drain_pos(0)
            for k in range(2 * B):
                slot = k % 4
                pslot = k // B
                osl = (k + 2) % 4

                if not (first and k < 2):
                    drain_out(osl)

                if not (last and k >= 2 * B - 2):
                    start_in((k + 2) % B, row0 + ((k + 2) // B) * T, osl)

                drain_in(slot)

                tref = tv[slot]
                pref = pv[pslot]

                for r in range(T):
                    @plsc.parallel_loop(0, D, step=L, unroll=8)
                    def _add(i):
                        plsc.addupdate(tref.at[r, pl.ds(i, L)],
                                       pref[r, pl.ds(i, L)])

                start_out(k % B, row0 + (k // B) * T, slot)

                if k == B - 1:
                    drain_pos(1)
                    if not last:
                        start_pos(row0 + 2 * T, 0)

            if not last:
                start_pos(row0 + 3 * T, 1)

        pair_body(0, True, False)

        @pl.loop(1, n_pairs - 1)
        def _pair(h):
            pair_body(h, False, False)

        pair_body(n_pairs - 1, False, True)

        drain_out(2)
        drain_out(3)

    return sc_kernel


@jax.jit
def kernel(token_embeddings, pos_table):
    B, S, D = token_embeddings.shape
    return _make_sc_kernel(B, S, D)(token_embeddings, pos_table[:S])


# speedup vs baseline: 1.0559x; 1.0559x over previous
import functools

import jax
import jax.numpy as jnp
from jax import lax
from jax.experimental import pallas as pl
from jax.experimental.pallas import tpu as pltpu
from jax.experimental.pallas import tpu_sc as plsc

NC = 2
NS = 16
NW = NC * NS
L = 16


def _make_sc_kernel(B, S, D):
    rows_w = S // NW
    T = 8
    n_chunks = rows_w // T
    n_pairs = n_chunks // 2

    mesh = plsc.VectorSubcoreMesh(core_axis_name="c", subcore_axis_name="s")

    @functools.partial(
        pl.kernel,
        out_type=jax.ShapeDtypeStruct((B, S, D), jnp.float32),
        mesh=mesh,
        scratch_types=[
            [pltpu.VMEM((T, D), jnp.float32)] * 4,
            [pltpu.VMEM((T, D), jnp.float32)] * 2,
            [pltpu.SemaphoreType.DMA] * 4,
            [pltpu.SemaphoreType.DMA] * 4,
            [pltpu.SemaphoreType.DMA] * 2,
        ],
    )
    def sc_kernel(tok_hbm, pos_hbm, out_hbm, tv, pv, sin, sout, spos):
        wid = lax.axis_index("s") * NC + lax.axis_index("c")
        s0 = wid * rows_w

        def start_in(b, row, slot):
            pltpu.async_copy(tok_hbm.at[b, pl.ds(row, T), :], tv[slot],
                             sin[slot])

        def drain_in(slot):
            pltpu.make_async_copy(tok_hbm.at[0, pl.ds(0, T), :], tv[slot],
                                  sin[slot]).wait()

        def start_out(b, row, slot):
            pltpu.async_copy(tv[slot], out_hbm.at[b, pl.ds(row, T), :],
                             sout[slot])

        def drain_out(slot):
            pltpu.make_async_copy(tv[slot], out_hbm.at[0, pl.ds(0, T), :],
                                  sout[slot]).wait()

        def start_pos(row, pslot):
            pltpu.async_copy(pos_hbm.at[pl.ds(row, T), :], pv[pslot],
                             spos[pslot])

        def drain_pos(pslot):
            pltpu.make_async_copy(pos_hbm.at[pl.ds(0, T), :], pv[pslot],
                                  spos[pslot]).wait()

        start_in(0, s0, 0)
        start_in(1, s0, 1)
        start_pos(s0, 0)
        start_pos(s0 + T, 1)

        @pl.loop(0, n_pairs)
        def _pair(h):
            row0 = s0 + h * (2 * T)
            not_first = h > 0
            not_last = h < n_pairs - 1
            drain_pos(0)
            for k in range(2 * B):
                slot = k % 4
                pslot = k // B
                osl = (k + 2) % 4

                if k < 2:
                    @pl.when(not_first)
                    def _():
                        drain_out(osl)
                else:
                    drain_out(osl)

                def _refill(k=k, osl=osl):
                    start_in((k + 2) % B, row0 + ((k + 2) // B) * T, osl)

                if k >= 2 * B - 2:
                    pl.when(not_last)(_refill)
                else:
                    _refill()

                drain_in(slot)

                tref = tv[slot]
                pref = pv[pslot]

                for r in range(T):
                    @plsc.parallel_loop(0, D, step=L, unroll=8)
                    def _add(i):
                        plsc.addupdate(tref.at[r, pl.ds(i, L)],
                                       pref[r, pl.ds(i, L)])

                start_out(k % B, row0 + (k // B) * T, slot)

                if k == B - 1:
                    drain_pos(1)

                    @pl.when(not_last)
                    def _():
                        start_pos(row0 + 2 * T, 0)

            @pl.when(not_last)
            def _():
                start_pos(row0 + 3 * T, 1)

        drain_out(2)
        drain_out(3)

    return sc_kernel


@jax.jit
def kernel(token_embeddings, pos_table):
    B, S, D = token_embeddings.shape
    return _make_sc_kernel(B, S, D)(token_embeddings, pos_table[:S])


# speedup vs baseline: 1.0880x; 1.0304x over previous
import functools

import jax
import jax.numpy as jnp
from jax import lax
from jax.experimental import pallas as pl
from jax.experimental.pallas import tpu as pltpu
from jax.experimental.pallas import tpu_sc as plsc

NC = 2
NS = 16
NW = NC * NS
L = 16


def _make_sc_kernel(B, S, D):
    rows_w = S // NW
    T = 4
    n_chunks = rows_w // T
    n_pairs = n_chunks // 2
    NB = 2 * B

    mesh = plsc.VectorSubcoreMesh(core_axis_name="c", subcore_axis_name="s")

    @functools.partial(
        pl.kernel,
        out_type=jax.ShapeDtypeStruct((B, S, D), jnp.float32),
        mesh=mesh,
        scratch_types=[
            [pltpu.VMEM((T, D), jnp.float32)] * NB,
            [pltpu.VMEM((T, D), jnp.float32)] * 2,
            [pltpu.SemaphoreType.DMA] * NB,
            [pltpu.SemaphoreType.DMA] * NB,
            [pltpu.SemaphoreType.DMA] * 2,
        ],
    )
    def sc_kernel(tok_hbm, pos_hbm, out_hbm, tv, pv, sin, sout, spos):
        wid = lax.axis_index("s") * NC + lax.axis_index("c")
        s0 = wid * rows_w

        def start_in(b, row, slot):
            pltpu.async_copy(tok_hbm.at[b, pl.ds(row, T), :], tv[slot],
                             sin[slot])

        def drain_in(slot):
            pltpu.make_async_copy(tok_hbm.at[0, pl.ds(0, T), :], tv[slot],
                                  sin[slot]).wait()

        def start_out(b, row, slot):
            pltpu.async_copy(tv[slot], out_hbm.at[b, pl.ds(row, T), :],
                             sout[slot])

        def drain_out(slot):
            pltpu.make_async_copy(tv[slot], out_hbm.at[0, pl.ds(0, T), :],
                                  sout[slot]).wait()

        def start_pos(row, pslot):
            pltpu.async_copy(pos_hbm.at[pl.ds(row, T), :], pv[pslot],
                             spos[pslot])

        def drain_pos(pslot):
            pltpu.make_async_copy(pos_hbm.at[pl.ds(0, T), :], pv[pslot],
                                  spos[pslot]).wait()

        for b in range(4):
            start_in(b, s0, b)
        start_pos(s0, 0)
        start_pos(s0 + T, 1)

        @pl.loop(0, n_pairs)
        def _pair(h):
            row0 = s0 + h * (2 * T)
            not_first = h > 0
            not_last = h < n_pairs - 1
            drain_pos(0)
            for k in range(NB):
                slot = k
                pslot = k // B
                osl = (k + 4) % NB

                if k < 4:
                    @pl.when(not_first)
                    def _():
                        drain_out(osl)
                else:
                    drain_out(osl)

                def _refill(k=k, osl=osl):
                    start_in((k + 4) % B, row0 + ((k + 4) // B) * T, osl)

                if k >= 4:
                    pl.when(not_last)(_refill)
                else:
                    _refill()

                drain_in(slot)

                tref = tv[slot]
                pref = pv[pslot]

                for r in range(T):
                    @plsc.parallel_loop(0, D, step=L, unroll=8)
                    def _add(i):
                        plsc.addupdate(tref.at[r, pl.ds(i, L)],
                                       pref[r, pl.ds(i, L)])

                start_out(k % B, row0 + (k // B) * T, slot)

                if k == B - 1:
                    drain_pos(1)

                    @pl.when(not_last)
                    def _():
                        start_pos(row0 + 2 * T, 0)

            @pl.when(not_last)
            def _():
                start_pos(row0 + 3 * T, 1)

        drain_out(4)
        drain_out(5)
        drain_out(6)
        drain_out(7)

    return sc_kernel


@jax.jit
def kernel(token_embeddings, pos_table):
    B, S, D = token_embeddings.shape
    return _make_sc_kernel(B, S, D)(token_embeddings, pos_table[:S])
